# baseline (speedup 1.0000x reference)
import functools

import jax
import jax.numpy as jnp
from jax import lax
from jax.experimental import pallas as pl
from jax.experimental.pallas import tpu as pltpu
from jax.experimental.pallas import tpu_sc as plsc

N = 102400
G = 512
D = 256
H = 512
OUT = 6
OUTP = 128

NC = 2
NS = 16
NW = NC * NS
L = 16
ROWS = N // G
SPW = G // NW
CHUNKS = D // L

_MESH = plsc.VectorSubcoreMesh(
    core_axis_name="c", subcore_axis_name="s", num_cores=NC, num_subcores=NS
)


def _seg_sum_body(nf_hbm, out_hbm, buf, acc, sem0, sem1):
    wid = lax.axis_index("s") * NC + lax.axis_index("c")
    seg0 = wid * SPW
    sems = (sem0, sem1)

    def start(s):
        return pltpu.async_copy(
            nf_hbm.at[pl.ds((seg0 + s) * ROWS, ROWS)], buf.at[s % 2], sems[s % 2]
        )

    cp = start(0)
    for s in range(SPW):
        cp.wait()
        if s + 1 < SPW:
            cp = start(s + 1)
        bi = s % 2

        def body(it, carry):
            r = it * 2
            half = tuple(
                buf[bi, r, pl.ds(c * L, L)] + buf[bi, r + 1, pl.ds(c * L, L)]
                for c in range(CHUNKS)
            )
            return tuple(carry[c] + half[c] for c in range(CHUNKS))

        zeros = tuple(jnp.zeros((L,), jnp.float32) for _ in range(CHUNKS))
        total = lax.fori_loop(0, ROWS // 2, body, zeros)
        for c in range(CHUNKS):
            acc[s, pl.ds(c * L, L)] = total[c]

    pltpu.sync_copy(acc, out_hbm.at[pl.ds(seg0, SPW)])


_seg_sum = functools.partial(
    pl.kernel,
    mesh=_MESH,
    out_type=jax.ShapeDtypeStruct((G, D), jnp.float32),
    scratch_types=[
        pltpu.VMEM((2, ROWS, D), jnp.float32),
        pltpu.VMEM((SPW, D), jnp.float32),
        pltpu.SemaphoreType.DMA,
        pltpu.SemaphoreType.DMA,
    ],
)(_seg_sum_body)


def _ssp(x):
    return jnp.maximum(x, 0.0) + jnp.log1p(jnp.exp(-jnp.abs(x))) - jnp.log(2.0)


def _mlp_body(x_ref, nn_ref, w0_ref, b0_ref, w1_ref, b1_ref, w2_ref, b2_ref, o_ref):
    inv = 1.0 / jnp.maximum(nn_ref[...], 1).astype(jnp.float32)
    x = x_ref[...] * inv
    h = _ssp(
        jnp.dot(x, w0_ref[...], preferred_element_type=jnp.float32,
                precision=lax.Precision.HIGHEST) + b0_ref[...]
    )
    h = _ssp(
        jnp.dot(h, w1_ref[...], preferred_element_type=jnp.float32,
                precision=lax.Precision.HIGHEST) + b1_ref[...]
    )
    o_ref[...] = (
        jnp.dot(h, w2_ref[...], preferred_element_type=jnp.float32,
                precision=lax.Precision.HIGHEST) + b2_ref[...]
    )


_mlp = pl.pallas_call(
    _mlp_body,
    out_shape=jax.ShapeDtypeStruct((G, OUT), jnp.float32),
)


@jax.jit
def kernel(node_features, n_node, W0, b0, W1, b1, W2, b2):
    sums = _seg_sum(node_features)
    return _mlp(sums, n_node[:, None], W0, b0[None, :], W1, b1[None, :],
                W2, b2[None, :])


# speedup vs baseline: 1.2476x; 1.2476x over previous
import functools

import jax
import jax.numpy as jnp
from jax import lax
from jax.experimental import pallas as pl
from jax.experimental.pallas import tpu as pltpu
from jax.experimental.pallas import tpu_sc as plsc

N = 102400
G = 512
D = 256
H = 512
OUT = 6

NC = 2
NS = 16
NW = NC * NS
L = 16
ROWS = N // G
CHUNKS = D // L

SC_SEGS = 256
TC_SEGS = G - SC_SEGS
SPW = SC_SEGS // NW
SPS = 8

_MESH = plsc.VectorSubcoreMesh(
    core_axis_name="c", subcore_axis_name="s", num_cores=NC, num_subcores=NS
)


def _seg_sum_body(nf_hbm, out_hbm, buf, acc, sem0, sem1):
    wid = lax.axis_index("s") * NC + lax.axis_index("c")
    seg0 = wid * SPW
    sems = (sem0, sem1)

    def start(s):
        return pltpu.async_copy(
            nf_hbm.at[pl.ds((seg0 + s) * ROWS, ROWS)], buf.at[s % 2], sems[s % 2]
        )

    cp = start(0)
    for s in range(SPW):
        cp.wait()
        if s + 1 < SPW:
            cp = start(s + 1)
        bi = s % 2

        def body(it, carry):
            r = it * 2
            half = tuple(
                buf[bi, r, pl.ds(c * L, L)] + buf[bi, r + 1, pl.ds(c * L, L)]
                for c in range(CHUNKS)
            )
            return tuple(carry[c] + half[c] for c in range(CHUNKS))

        zeros = tuple(jnp.zeros((L,), jnp.float32) for _ in range(CHUNKS))
        total = lax.fori_loop(0, ROWS // 2, body, zeros)
        for c in range(CHUNKS):
            acc[s, pl.ds(c * L, L)] = total[c]

    pltpu.sync_copy(acc, out_hbm.at[pl.ds(seg0, SPW)])


_seg_sum = functools.partial(
    pl.kernel,
    mesh=_MESH,
    out_type=jax.ShapeDtypeStruct((SC_SEGS, D), jnp.float32),
    scratch_types=[
        pltpu.VMEM((2, ROWS, D), jnp.float32),
        pltpu.VMEM((SPW, D), jnp.float32),
        pltpu.SemaphoreType.DMA,
        pltpu.SemaphoreType.DMA,
    ],
)(_seg_sum_body)


def _tc_reduce_body(x_ref, o_ref):
    for s in range(SPS):
        o_ref[s, :] = jnp.sum(x_ref[pl.ds(s * ROWS, ROWS), :], axis=0)


_tc_reduce = pl.pallas_call(
    _tc_reduce_body,
    grid=(TC_SEGS // SPS,),
    in_specs=[
        pl.BlockSpec((SPS * ROWS, D), lambda i: (i + SC_SEGS // SPS, 0)),
    ],
    out_specs=pl.BlockSpec((SPS, D), lambda i: (i, 0)),
    out_shape=jax.ShapeDtypeStruct((TC_SEGS, D), jnp.float32),
)


def _ssp(x):
    return jnp.maximum(x, 0.0) + jnp.log1p(jnp.exp(-jnp.abs(x))) - jnp.log(2.0)


def _mlp_body(xa_ref, xb_ref, nn_ref, w0_ref, b0_ref, w1_ref, b1_ref,
              w2_ref, b2_ref, o_ref):
    inv = 1.0 / jnp.maximum(nn_ref[...], 1).astype(jnp.float32)
    x = jnp.concatenate([xa_ref[...], xb_ref[...]], axis=0) * inv
    h = _ssp(
        jnp.dot(x, w0_ref[...], preferred_element_type=jnp.float32,
                precision=lax.Precision.HIGHEST) + b0_ref[...]
    )
    h = _ssp(
        jnp.dot(h, w1_ref[...], preferred_element_type=jnp.float32,
                precision=lax.Precision.HIGHEST) + b1_ref[...]
    )
    o_ref[...] = (
        jnp.dot(h, w2_ref[...], preferred_element_type=jnp.float32,
                precision=lax.Precision.HIGHEST) + b2_ref[...]
    )


_mlp = pl.pallas_call(
    _mlp_body,
    out_shape=jax.ShapeDtypeStruct((G, OUT), jnp.float32),
)


@jax.jit
def kernel(node_features, n_node, W0, b0, W1, b1, W2, b2):
    sc_sums = _seg_sum(node_features)
    tc_sums = _tc_reduce(node_features)
    return _mlp(sc_sums, tc_sums, n_node[:, None], W0, b0[None, :],
                W1, b1[None, :], W2, b2[None, :])
